# baseline (speedup 1.0000x reference)
import functools

import jax
import jax.numpy as jnp
from jax import lax
from jax.experimental import pallas as pl
from jax.experimental.pallas import tpu as pltpu
from jax.experimental.pallas import tpu_sc as plsc

N_NODES = 10000
N_EDGES = 320000
D = 128

NC = 2
NS = 16
L = 16

CHUNK = 128
NCH = 80
N_CHUNKS = NC * NS * NCH
E_PAD = N_CHUNKS * CHUNK

ROWS_PER_TILE = 624
REM_BASE = ROWS_PER_TILE * NS
REM_ROWS = N_NODES - REM_BASE

_mesh = plsc.VectorSubcoreMesh(core_axis_name="c", subcore_axis_name="s")


@functools.partial(
    pl.kernel,
    out_type=jax.ShapeDtypeStruct((NC, N_NODES, D), jnp.float32),
    mesh=_mesh,
    compiler_params=pltpu.CompilerParams(needs_layout_passes=False),
    scratch_types=[
        pltpu.VMEM_SHARED((N_NODES, D), jnp.float32),
        pltpu.VMEM((CHUNK,), jnp.int32),
        pltpu.VMEM((CHUNK,), jnp.int32),
        pltpu.VMEM((CHUNK,), jnp.int32),
        pltpu.VMEM((CHUNK,), jnp.int32),
        pltpu.VMEM((CHUNK,), jnp.float32),
        pltpu.VMEM((CHUNK,), jnp.float32),
        pltpu.VMEM((CHUNK, D), jnp.float32),
        pltpu.VMEM((CHUNK, D), jnp.float32),
        pltpu.SemaphoreType.DMA,
        pltpu.SemaphoreType.DMA,
        pltpu.SemaphoreType.DMA,
        pltpu.SemaphoreType.DMA,
        pltpu.SemaphoreType.DMA,
        pltpu.SemaphoreType.DMA,
        pltpu.SemaphoreType.DMA,
        pltpu.SemaphoreType.DMA,
    ],
)
def _sc_agg(h_hbm, src_hbm, dst_hbm, w_hbm, zeros_hbm, out_hbm,
            acc, sv0, sv1, dv0, dv1, wv0, wv1, rb0, rb1,
            ssr0, ssr1, sds0, sds1, sw0, sw1, sg0, sg1):
    cid = lax.axis_index("c")
    sid = lax.axis_index("s")
    wid = sid * NC + cid
    first = wid * NCH

    svs = (sv0, sv1)
    dvs = (dv0, dv1)
    wvs = (wv0, wv1)
    rbs = (rb0, rb1)
    ssrs = (ssr0, ssr1)
    sdss = (sds0, sds1)
    sws = (sw0, sw1)
    sgs = (sg0, sg1)

    pltpu.sync_copy(zeros_hbm.at[pl.ds(sid * ROWS_PER_TILE, ROWS_PER_TILE)],
                    acc.at[pl.ds(sid * ROWS_PER_TILE, ROWS_PER_TILE)])

    @pl.when(sid == NS - 1)
    def _zero_rem():
        pltpu.sync_copy(zeros_hbm.at[pl.ds(REM_BASE, REM_ROWS)],
                        acc.at[pl.ds(REM_BASE, REM_ROWS)])

    plsc.subcore_barrier()

    def idx_fire(c, p):
        off = (first + c) * CHUNK
        pltpu.async_copy(src_hbm.at[pl.ds(off, CHUNK)], svs[p], ssrs[p])
        pltpu.async_copy(dst_hbm.at[pl.ds(off, CHUNK)], dvs[p], sdss[p])
        pltpu.async_copy(w_hbm.at[pl.ds(off, CHUNK)], wvs[p], sws[p])

    def idx_drain(p):
        pltpu.make_async_copy(src_hbm.at[pl.ds(0, CHUNK)], svs[p], ssrs[p]).wait()
        pltpu.make_async_copy(dst_hbm.at[pl.ds(0, CHUNK)], dvs[p], sdss[p]).wait()
        pltpu.make_async_copy(w_hbm.at[pl.ds(0, CHUNK)], wvs[p], sws[p]).wait()

    def gather_fire(p):
        pltpu.async_copy(h_hbm.at[svs[p]], rbs[p], sgs[p])

    def gather_drain(p):
        pltpu.make_async_copy(h_hbm.at[svs[p]], rbs[p], sgs[p]).wait()

    def scale(p):
        def sbody(e, c2):
            w16 = plsc.load_gather(wvs[p], [jnp.broadcast_to(e, (L,))])
            for j in range(D // L):
                sl = pl.ds(j * L, L)
                rbs[p][e, sl] = rbs[p][e, sl] * w16
            return c2

        lax.fori_loop(0, CHUNK, sbody, 0, unroll=2)

    def scatter(p):
        pltpu.sync_copy(rbs[p], acc.at[dvs[p]], add=True)

    def body(c, p, fire_next=True):
        if fire_next:
            idx_fire(c + 1, 1 - p)
        gather_drain(p)
        scale(p)
        if fire_next:
            idx_drain(1 - p)
            gather_fire(1 - p)
        scatter(p)

    idx_fire(0, 0)
    idx_drain(0)
    gather_fire(0)

    def block(j, carry):
        c = 2 * j
        body(c, 0)
        body(c + 1, 1)
        return carry

    lax.fori_loop(0, NCH // 2 - 1, block, 0)

    body(NCH - 2, 0)
    body(NCH - 1, 1, fire_next=False)

    plsc.subcore_barrier()

    pltpu.sync_copy(acc.at[pl.ds(sid * ROWS_PER_TILE, ROWS_PER_TILE)],
                    out_hbm.at[cid, pl.ds(sid * ROWS_PER_TILE, ROWS_PER_TILE)])

    @pl.when(sid == NS - 1)
    def _write_rem():
        pltpu.sync_copy(acc.at[pl.ds(REM_BASE, REM_ROWS)],
                        out_hbm.at[cid, pl.ds(REM_BASE, REM_ROWS)])


_BLK = 1000


def _tc_body(p_ref, h_ref, wrel_ref, wroot_ref, b_ref, o_ref):
    agg = p_ref[0] + p_ref[1]
    o_ref[...] = (
        jnp.dot(agg, wrel_ref[...], preferred_element_type=jnp.float32)
        + jnp.dot(h_ref[...], wroot_ref[...], preferred_element_type=jnp.float32)
        + b_ref[...]
    )


_tc_combine = pl.pallas_call(
    _tc_body,
    grid=(N_NODES // _BLK,),
    in_specs=[
        pl.BlockSpec((NC, _BLK, D), lambda i: (0, i, 0)),
        pl.BlockSpec((_BLK, D), lambda i: (i, 0)),
        pl.BlockSpec((D, D), lambda i: (0, 0)),
        pl.BlockSpec((D, D), lambda i: (0, 0)),
        pl.BlockSpec((1, D), lambda i: (0, 0)),
    ],
    out_specs=pl.BlockSpec((_BLK, D), lambda i: (i, 0)),
    out_shape=jax.ShapeDtypeStruct((N_NODES, D), jnp.float32),
)


def kernel(x, edge_index, edge_attr, W_rel1, b_rel1, W_root1,
           W_rel2, b_rel2, W_root2):
    pad = E_PAD - N_EDGES
    src = jnp.concatenate([edge_index[0], jnp.zeros((pad,), jnp.int32)])
    dst = jnp.concatenate([edge_index[1], jnp.zeros((pad,), jnp.int32)])
    w = jnp.concatenate([edge_attr, jnp.zeros((pad,), jnp.float32)])
    zeros = jnp.zeros((N_NODES, D), jnp.float32)

    p1 = _sc_agg(x, src, dst, w, zeros)
    h1 = _tc_combine(p1, x, W_rel1, W_root1, b_rel1.reshape(1, D))
    p2 = _sc_agg(h1, src, dst, w, zeros)
    h2 = _tc_combine(p2, h1, W_rel2, W_root2, b_rel2.reshape(1, D))
    return h2


# speedup vs baseline: 2.0439x; 2.0439x over previous
import functools

import jax
import jax.numpy as jnp
from jax import lax
from jax.experimental import pallas as pl
from jax.experimental.pallas import tpu as pltpu
from jax.experimental.pallas import tpu_sc as plsc

N_NODES = 10000
N_EDGES = 320000
D = 128

NC = 2
NS = 16
L = 16

CHUNK = 128
HALF = CHUNK // 2
N_CHUNKS = N_EDGES // CHUNK
ROWS_PER_TILE = 624
REM_BASE = ROWS_PER_TILE * NS
REM_ROWS = N_NODES - REM_BASE

_mesh = plsc.VectorSubcoreMesh(core_axis_name="c", subcore_axis_name="s")


@functools.partial(
    pl.kernel,
    out_type=jax.ShapeDtypeStruct((NC, N_NODES, D), jnp.float32),
    mesh=_mesh,
    compiler_params=pltpu.CompilerParams(needs_layout_passes=False),
    scratch_types=[
        pltpu.VMEM_SHARED((N_NODES, D), jnp.float32),
        pltpu.VMEM((HALF,), jnp.int32),
        pltpu.VMEM((HALF,), jnp.int32),
        pltpu.VMEM((HALF,), jnp.int32),
        pltpu.VMEM((HALF,), jnp.int32),
        pltpu.VMEM((CHUNK,), jnp.float32),
        pltpu.VMEM((CHUNK, D), jnp.float32),
        pltpu.SemaphoreType.DMA,
        pltpu.SemaphoreType.DMA,
        pltpu.SemaphoreType.DMA,
        pltpu.SemaphoreType.DMA,
        pltpu.SemaphoreType.DMA,
        pltpu.SemaphoreType.DMA,
        pltpu.SemaphoreType.DMA,
        pltpu.SemaphoreType.DMA,
        pltpu.SemaphoreType.DMA,
    ],
)
def _sc_agg(h_hbm, src_hbm, dst_hbm, w_hbm, zeros_hbm, out_hbm,
            acc, sva, svb, dva, dvb, w_v, rows_v,
            s_sa, s_sb, s_da, s_db, s_w, s_ga, s_gb, s_ta, s_tb):
    cid = lax.axis_index("c")
    sid = lax.axis_index("s")
    wid = sid * NC + cid

    pltpu.sync_copy(zeros_hbm.at[pl.ds(sid * ROWS_PER_TILE, ROWS_PER_TILE)],
                    acc.at[pl.ds(sid * ROWS_PER_TILE, ROWS_PER_TILE)])

    @pl.when(sid == NS - 1)
    def _zero_rem():
        pltpu.sync_copy(zeros_hbm.at[pl.ds(REM_BASE, REM_ROWS)],
                        acc.at[pl.ds(REM_BASE, REM_ROWS)])

    plsc.subcore_barrier()

    n_base = N_CHUNKS // (NC * NS)
    n_rem = N_CHUNKS % (NC * NS)
    my_n = jnp.where(wid < n_rem, n_base + 1, n_base)
    my_start = wid * n_base + jnp.minimum(wid, n_rem)

    def scale_half(lo):
        def scale_body(e, c2):
            w16 = plsc.load_gather(w_v, [jnp.broadcast_to(e, (L,))])
            for j in range(D // L):
                sl = pl.ds(j * L, L)
                rows_v[e, sl] = rows_v[e, sl] * w16
            return c2

        lax.fori_loop(lo, lo + HALF, scale_body, 0, unroll=2)

    def chunk_body(g, carry):
        base = (my_start + g) * CHUNK
        d_sa = pltpu.async_copy(src_hbm.at[pl.ds(base, HALF)], sva, s_sa)
        d_sb = pltpu.async_copy(src_hbm.at[pl.ds(base + HALF, HALF)], svb, s_sb)
        d_da = pltpu.async_copy(dst_hbm.at[pl.ds(base, HALF)], dva, s_da)
        d_db = pltpu.async_copy(dst_hbm.at[pl.ds(base + HALF, HALF)], dvb, s_db)
        d_w = pltpu.async_copy(w_hbm.at[pl.ds(base, CHUNK)], w_v, s_w)
        d_sa.wait()
        d_ga = pltpu.async_copy(h_hbm.at[sva], rows_v.at[pl.ds(0, HALF)], s_ga)
        d_sb.wait()
        d_gb = pltpu.async_copy(h_hbm.at[svb], rows_v.at[pl.ds(HALF, HALF)],
                                s_gb)
        d_w.wait()
        d_ga.wait()
        scale_half(0)
        d_da.wait()
        d_ta = pltpu.async_copy(rows_v.at[pl.ds(0, HALF)], acc.at[dva], s_ta,
                                add=True)
        d_gb.wait()
        scale_half(HALF)
        d_db.wait()
        d_tb = pltpu.async_copy(rows_v.at[pl.ds(HALF, HALF)], acc.at[dvb],
                                s_tb, add=True)
        d_ta.wait()
        d_tb.wait()
        return carry

    lax.fori_loop(0, my_n, chunk_body, 0)
    plsc.subcore_barrier()

    pltpu.sync_copy(acc.at[pl.ds(sid * ROWS_PER_TILE, ROWS_PER_TILE)],
                    out_hbm.at[cid, pl.ds(sid * ROWS_PER_TILE, ROWS_PER_TILE)])

    @pl.when(sid == NS - 1)
    def _write_rem():
        pltpu.sync_copy(acc.at[pl.ds(REM_BASE, REM_ROWS)],
                        out_hbm.at[cid, pl.ds(REM_BASE, REM_ROWS)])


_BLK = 1000


def _tc_body(p_ref, h_ref, wrel_ref, wroot_ref, b_ref, o_ref):
    agg = p_ref[0] + p_ref[1]
    o_ref[...] = (
        jnp.dot(agg, wrel_ref[...], preferred_element_type=jnp.float32)
        + jnp.dot(h_ref[...], wroot_ref[...], preferred_element_type=jnp.float32)
        + b_ref[...]
    )


_tc_combine = pl.pallas_call(
    _tc_body,
    grid=(N_NODES // _BLK,),
    in_specs=[
        pl.BlockSpec((NC, _BLK, D), lambda i: (0, i, 0)),
        pl.BlockSpec((_BLK, D), lambda i: (i, 0)),
        pl.BlockSpec((D, D), lambda i: (0, 0)),
        pl.BlockSpec((D, D), lambda i: (0, 0)),
        pl.BlockSpec((1, D), lambda i: (0, 0)),
    ],
    out_specs=pl.BlockSpec((_BLK, D), lambda i: (i, 0)),
    out_shape=jax.ShapeDtypeStruct((N_NODES, D), jnp.float32),
)


def kernel(x, edge_index, edge_attr, W_rel1, b_rel1, W_root1,
           W_rel2, b_rel2, W_root2):
    src = edge_index[0]
    dst = edge_index[1]
    zeros = jnp.zeros((N_NODES, D), jnp.float32)

    p1 = _sc_agg(x, src, dst, edge_attr, zeros)
    h1 = _tc_combine(p1, x, W_rel1, W_root1, b_rel1.reshape(1, D))
    p2 = _sc_agg(h1, src, dst, edge_attr, zeros)
    h2 = _tc_combine(p2, h1, W_rel2, W_root2, b_rel2.reshape(1, D))
    return h2


# speedup vs baseline: 2.1648x; 1.0591x over previous
import functools

import jax
import jax.numpy as jnp
from jax import lax
from jax.experimental import pallas as pl
from jax.experimental.pallas import tpu as pltpu
from jax.experimental.pallas import tpu_sc as plsc

N_NODES = 10000
N_EDGES = 320000
D = 128

NC = 2
NS = 16
L = 16

CHUNK = 128
HALF = CHUNK // 2
Q1 = 32
N_CHUNKS = N_EDGES // CHUNK
ROWS_PER_TILE = 624
REM_BASE = ROWS_PER_TILE * NS
REM_ROWS = N_NODES - REM_BASE

_mesh = plsc.VectorSubcoreMesh(core_axis_name="c", subcore_axis_name="s")


@functools.partial(
    pl.kernel,
    out_type=jax.ShapeDtypeStruct((NC, N_NODES, D), jnp.float32),
    mesh=_mesh,
    compiler_params=pltpu.CompilerParams(needs_layout_passes=False),
    scratch_types=[
        pltpu.VMEM_SHARED((N_NODES, D), jnp.float32),
        pltpu.VMEM((Q1,), jnp.int32),
        pltpu.VMEM((CHUNK - Q1,), jnp.int32),
        pltpu.VMEM((HALF,), jnp.int32),
        pltpu.VMEM((HALF,), jnp.int32),
        pltpu.VMEM((CHUNK,), jnp.float32),
        pltpu.VMEM((CHUNK, D), jnp.float32),
        pltpu.SemaphoreType.DMA,
        pltpu.SemaphoreType.DMA,
        pltpu.SemaphoreType.DMA,
        pltpu.SemaphoreType.DMA,
        pltpu.SemaphoreType.DMA,
        pltpu.SemaphoreType.DMA,
        pltpu.SemaphoreType.DMA,
        pltpu.SemaphoreType.DMA,
        pltpu.SemaphoreType.DMA,
    ],
)
def _sc_agg(h_hbm, src_hbm, dst_hbm, w_hbm, zeros_hbm, out_hbm,
            acc, sva, svb, dva, dvb, w_v, rows_v,
            s_sa, s_sb, s_da, s_db, s_w, s_ga, s_gb, s_ta, s_tb):
    cid = lax.axis_index("c")
    sid = lax.axis_index("s")
    wid = sid * NC + cid

    pltpu.sync_copy(zeros_hbm.at[pl.ds(sid * ROWS_PER_TILE, ROWS_PER_TILE)],
                    acc.at[pl.ds(sid * ROWS_PER_TILE, ROWS_PER_TILE)])

    @pl.when(sid == NS - 1)
    def _zero_rem():
        pltpu.sync_copy(zeros_hbm.at[pl.ds(REM_BASE, REM_ROWS)],
                        acc.at[pl.ds(REM_BASE, REM_ROWS)])

    plsc.subcore_barrier()

    n_base = N_CHUNKS // (NC * NS)
    n_rem = N_CHUNKS % (NC * NS)
    my_n = jnp.where(wid < n_rem, n_base + 1, n_base)
    my_start = wid * n_base + jnp.minimum(wid, n_rem)

    def scale_part(lo, hi):
        def scale_body(e, c2):
            w16 = plsc.load_gather(w_v, [jnp.broadcast_to(e, (L,))])
            for j in range(D // L):
                sl = pl.ds(j * L, L)
                rows_v[e, sl] = rows_v[e, sl] * w16
            return c2

        lax.fori_loop(lo, hi, scale_body, 0, unroll=4)

    def chunk_body(g, carry):
        base = (my_start + g) * CHUNK
        d_sa = pltpu.async_copy(src_hbm.at[pl.ds(base, Q1)], sva, s_sa)
        d_sb = pltpu.async_copy(src_hbm.at[pl.ds(base + Q1, CHUNK - Q1)], svb,
                                s_sb)
        d_da = pltpu.async_copy(dst_hbm.at[pl.ds(base, HALF)], dva, s_da)
        d_db = pltpu.async_copy(dst_hbm.at[pl.ds(base + HALF, HALF)], dvb, s_db)
        d_w = pltpu.async_copy(w_hbm.at[pl.ds(base, CHUNK)], w_v, s_w)
        d_sa.wait()
        d_ga = pltpu.async_copy(h_hbm.at[sva], rows_v.at[pl.ds(0, Q1)], s_ga)
        d_sb.wait()
        d_gb = pltpu.async_copy(h_hbm.at[svb],
                                rows_v.at[pl.ds(Q1, CHUNK - Q1)], s_gb)
        d_w.wait()
        d_ga.wait()
        scale_part(0, Q1)
        d_gb.wait()
        scale_part(Q1, HALF)
        d_da.wait()
        d_ta = pltpu.async_copy(rows_v.at[pl.ds(0, HALF)], acc.at[dva], s_ta,
                                add=True)
        scale_part(HALF, CHUNK)
        d_db.wait()
        d_tb = pltpu.async_copy(rows_v.at[pl.ds(HALF, HALF)], acc.at[dvb],
                                s_tb, add=True)
        d_ta.wait()
        d_tb.wait()
        return carry

    lax.fori_loop(0, my_n, chunk_body, 0)
    plsc.subcore_barrier()

    pltpu.sync_copy(acc.at[pl.ds(sid * ROWS_PER_TILE, ROWS_PER_TILE)],
                    out_hbm.at[cid, pl.ds(sid * ROWS_PER_TILE, ROWS_PER_TILE)])

    @pl.when(sid == NS - 1)
    def _write_rem():
        pltpu.sync_copy(acc.at[pl.ds(REM_BASE, REM_ROWS)],
                        out_hbm.at[cid, pl.ds(REM_BASE, REM_ROWS)])


_BLK = 1000


def _tc_body(p_ref, h_ref, wrel_ref, wroot_ref, b_ref, o_ref):
    agg = p_ref[0] + p_ref[1]
    o_ref[...] = (
        jnp.dot(agg, wrel_ref[...], preferred_element_type=jnp.float32)
        + jnp.dot(h_ref[...], wroot_ref[...], preferred_element_type=jnp.float32)
        + b_ref[...]
    )


_tc_combine = pl.pallas_call(
    _tc_body,
    grid=(N_NODES // _BLK,),
    in_specs=[
        pl.BlockSpec((NC, _BLK, D), lambda i: (0, i, 0)),
        pl.BlockSpec((_BLK, D), lambda i: (i, 0)),
        pl.BlockSpec((D, D), lambda i: (0, 0)),
        pl.BlockSpec((D, D), lambda i: (0, 0)),
        pl.BlockSpec((1, D), lambda i: (0, 0)),
    ],
    out_specs=pl.BlockSpec((_BLK, D), lambda i: (i, 0)),
    out_shape=jax.ShapeDtypeStruct((N_NODES, D), jnp.float32),
)


def kernel(x, edge_index, edge_attr, W_rel1, b_rel1, W_root1,
           W_rel2, b_rel2, W_root2):
    src = edge_index[0]
    dst = edge_index[1]
    zeros = jnp.zeros((N_NODES, D), jnp.float32)

    p1 = _sc_agg(x, src, dst, edge_attr, zeros)
    h1 = _tc_combine(p1, x, W_rel1, W_root1, b_rel1.reshape(1, D))
    p2 = _sc_agg(h1, src, dst, edge_attr, zeros)
    h2 = _tc_combine(p2, h1, W_rel2, W_root2, b_rel2.reshape(1, D))
    return h2
